# baseline (speedup 1.0000x reference)
import functools

import jax
import jax.numpy as jnp
from jax import lax
from jax.experimental import pallas as pl
from jax.experimental.pallas import tpu as pltpu
from jax.experimental.pallas import tpu_sc as plsc

_N = 8192
_D = 4096
_TH = 0.5
_BC = 512
_NB = _N // _BC
_L = 16


def _tc_body(x_ref, w_ref, mask_ref, logits_ref):
    i = pl.program_id(0)
    blk = jnp.dot(x_ref[...], w_ref[...], preferred_element_type=jnp.float32)
    logits_ref[:, pl.ds(i * _BC, _BC)] = blk

    @pl.when(i == _NB - 1)
    def _():
        lg = logits_ref[...]
        m = jnp.max(lg)
        e = jnp.exp(lg - m)
        s = jnp.sum(e)
        scores = e / s
        mask_ref[...] = (scores > (_TH / _N)).astype(jnp.int32)


_tc_mask = pl.pallas_call(
    _tc_body,
    grid=(_NB,),
    in_specs=[
        pl.BlockSpec((1, _D), lambda i: (0, 0)),
        pl.BlockSpec((_D, _BC), lambda i: (0, i)),
    ],
    out_specs=pl.BlockSpec((1, _N), lambda i: (0, 0)),
    out_shape=jax.ShapeDtypeStruct((1, _N), jnp.int32),
    scratch_shapes=[pltpu.VMEM((1, _N), jnp.float32)],
)


def _lane_gather(v, src):
    return lax.gather(
        v,
        src[:, None],
        lax.GatherDimensionNumbers(
            offset_dims=(), collapsed_slice_dims=(0,), start_index_map=(0,)
        ),
        slice_sizes=(1,),
        mode=lax.GatherScatterMode.PROMISE_IN_BOUNDS,
    )


def _lane_shift_down(v, k, lanes):
    shifted = _lane_gather(v, jnp.maximum(lanes - k, 0))
    return jnp.where(lanes >= k, shifted, 0)


_W = 16
_CW = _N // _W
_VW = _CW // _L


def _vreg_prefix(m, lanes):
    incl = m
    for k in (1, 2, 4, 8):
        incl = incl + _lane_shift_down(incl, k, lanes)
    return incl


def _sc_body(mask_hbm, out_hbm, mask_v, dest_v, val_v, cnt_v, call_v, counts_sh, sem):
    w = lax.axis_index("s")
    lanes = lax.iota(jnp.int32, _L)
    last = lanes * 0 + (_L - 1)

    pltpu.sync_copy(mask_hbm.at[pl.ds(w * _CW, _CW)], mask_v)

    cnt = jnp.zeros((_L,), jnp.int32)
    for j in range(_VW):
        m = mask_v[pl.ds(j * _L, _L)]
        incl = _vreg_prefix(m, lanes)
        cnt = cnt + _lane_gather(incl, last)
    cnt_v[...] = cnt
    pltpu.sync_copy(cnt_v.at[pl.ds(0, 8)], counts_sh.at[pl.ds(w * 8, 8)])
    plsc.subcore_barrier()

    pltpu.sync_copy(counts_sh, call_v)
    c = plsc.load_gather(call_v, [lanes * 8])
    cincl = _vreg_prefix(c, lanes)
    total = _lane_gather(cincl, last)
    base = _lane_gather(cincl - c, lanes * 0 + w)

    off = jnp.zeros((_L,), jnp.int32)
    for j in range(_VW):
        m = mask_v[pl.ds(j * _L, _L)]
        mb = m > 0
        incl = _vreg_prefix(m, lanes)
        gidx = w * _CW + j * _L + lanes
        pos = (base + off + incl) - m
        dest = jnp.where(mb, pos, (total + gidx) - pos)
        val = jnp.where(mb, gidx, -1)
        r, cc = divmod(j, 128 // _L)
        dest_v[r, pl.ds(cc * _L, _L)] = dest
        val_v[r, pl.ds(cc * _L, _L)] = val
        off = off + _lane_gather(incl, last)

    copies = [
        pltpu.async_copy(val_v.at[r], out_hbm.at[dest_v.at[r]], sem)
        for r in range(_CW // 128)
    ]
    for cp in copies:
        cp.wait()


@functools.cache
def _sc_compact():
    return pl.kernel(
        _sc_body,
        mesh=plsc.VectorSubcoreMesh(
            core_axis_name="c", subcore_axis_name="s", num_cores=1
        ),
        out_type=jax.ShapeDtypeStruct((_N,), jnp.int32),
        scratch_types=[
            pltpu.VMEM((_CW,), jnp.int32),
            pltpu.VMEM((_CW // 128, 128), jnp.int32),
            pltpu.VMEM((_CW // 128, 128), jnp.int32),
            pltpu.VMEM((_L,), jnp.int32),
            pltpu.VMEM((_W * 8,), jnp.int32),
            pltpu.VMEM_SHARED((_W * 8,), jnp.int32),
            pltpu.SemaphoreType.DMA,
        ],
        compiler_params=pltpu.CompilerParams(needs_layout_passes=False),
    )


def kernel(x, W):
    mask2d = _tc_mask(x.reshape(1, _D), W)
    inds = _sc_compact()(mask2d.reshape(_N))
    return inds, mask2d


# speedup vs baseline: 1.7711x; 1.7711x over previous
import functools

import jax
import jax.numpy as jnp
from jax import lax
from jax.experimental import pallas as pl
from jax.experimental.pallas import tpu as pltpu
from jax.experimental.pallas import tpu_sc as plsc

_N = 8192
_D = 4096
_TH = 0.5
_BC = 512
_NB = _N // _BC
_L = 16


def _tc_body(x_ref, w_ref, mask_ref, logits_ref):
    i = pl.program_id(0)
    blk = jnp.dot(x_ref[...], w_ref[...], preferred_element_type=jnp.float32)
    logits_ref[:, pl.ds(i * _BC, _BC)] = blk

    @pl.when(i == _NB - 1)
    def _():
        lg = logits_ref[...]
        m = jnp.max(lg)
        e = jnp.exp(lg - m)
        s = jnp.sum(e)
        scores = e / s
        mask_ref[...] = (scores > (_TH / _N)).astype(jnp.int32)


_tc_mask = pl.pallas_call(
    _tc_body,
    grid=(_NB,),
    in_specs=[
        pl.BlockSpec((1, _D), lambda i: (0, 0)),
        pl.BlockSpec((_D, _BC), lambda i: (0, i)),
    ],
    out_specs=pl.BlockSpec((1, _N), lambda i: (0, 0)),
    out_shape=jax.ShapeDtypeStruct((1, _N), jnp.int32),
    scratch_shapes=[pltpu.VMEM((1, _N), jnp.float32)],
)


def _lane_gather(v, src):
    return lax.gather(
        v,
        src[:, None],
        lax.GatherDimensionNumbers(
            offset_dims=(), collapsed_slice_dims=(0,), start_index_map=(0,)
        ),
        slice_sizes=(1,),
        mode=lax.GatherScatterMode.PROMISE_IN_BOUNDS,
    )


def _lane_shift_down(v, k, lanes):
    shifted = _lane_gather(v, jnp.maximum(lanes - k, 0))
    return jnp.where(lanes >= k, shifted, 0)


_W = 16
_CW = _N // _W
_VW = _CW // _L


def _vreg_prefix(m, lanes):
    incl = m
    for k in (1, 2, 4, 8):
        incl = incl + _lane_shift_down(incl, k, lanes)
    return incl


def _sc_body(mask_hbm, out_hbm, mask_v, dest_v, val_v, cnt_v, call_v, counts_sh, out_sh, sem):
    w = lax.axis_index("s")
    lanes = lax.iota(jnp.int32, _L)
    last = lanes * 0 + (_L - 1)

    pltpu.sync_copy(mask_hbm.at[pl.ds(w * _CW, _CW)], mask_v)

    cnt = jnp.zeros((_L,), jnp.int32)
    for j in range(_VW):
        m = mask_v[pl.ds(j * _L, _L)]
        incl = _vreg_prefix(m, lanes)
        cnt = cnt + _lane_gather(incl, last)
    cnt_v[...] = cnt
    pltpu.sync_copy(cnt_v.at[pl.ds(0, 8)], counts_sh.at[pl.ds(w * 8, 8)])
    plsc.subcore_barrier()

    pltpu.sync_copy(counts_sh, call_v)
    c = plsc.load_gather(call_v, [lanes * 8])
    cincl = _vreg_prefix(c, lanes)
    total = _lane_gather(cincl, last)
    base = _lane_gather(cincl - c, lanes * 0 + w)

    off = jnp.zeros((_L,), jnp.int32)
    for j in range(_VW):
        m = mask_v[pl.ds(j * _L, _L)]
        mb = m > 0
        incl = _vreg_prefix(m, lanes)
        gidx = w * _CW + j * _L + lanes
        pos = (base + off + incl) - m
        dest = jnp.where(mb, pos, (total + gidx) - pos)
        val = jnp.where(mb, gidx, -1)
        r, cc = divmod(j, 128 // _L)
        dest_v[r, pl.ds(cc * _L, _L)] = dest
        val_v[r, pl.ds(cc * _L, _L)] = val
        off = off + _lane_gather(incl, last)

    for r in range(_CW // 128):
        pltpu.sync_copy(val_v.at[r], out_sh.at[dest_v.at[r]])
    plsc.subcore_barrier()
    pltpu.sync_copy(out_sh.at[pl.ds(w * _CW, _CW)], out_hbm.at[pl.ds(w * _CW, _CW)])


@functools.cache
def _sc_compact():
    return pl.kernel(
        _sc_body,
        mesh=plsc.VectorSubcoreMesh(
            core_axis_name="c", subcore_axis_name="s", num_cores=1
        ),
        out_type=jax.ShapeDtypeStruct((_N,), jnp.int32),
        scratch_types=[
            pltpu.VMEM((_CW,), jnp.int32),
            pltpu.VMEM((_CW // 128, 128), jnp.int32),
            pltpu.VMEM((_CW // 128, 128), jnp.int32),
            pltpu.VMEM((_L,), jnp.int32),
            pltpu.VMEM((_W * 8,), jnp.int32),
            pltpu.VMEM_SHARED((_W * 8,), jnp.int32),
            pltpu.VMEM_SHARED((_N,), jnp.int32),
            pltpu.SemaphoreType.DMA,
        ],
        compiler_params=pltpu.CompilerParams(needs_layout_passes=False),
    )


def kernel(x, W):
    mask2d = _tc_mask(x.reshape(1, _D), W)
    inds = _sc_compact()(mask2d.reshape(_N))
    return inds, mask2d
